# Initial kernel scaffold; baseline (speedup 1.0000x reference)
import jax
import jax.numpy as jnp
from jax.experimental import pallas as pl


def kernel(x, edges, lin1_w, lin1_b, lin2_w, lin2_b, lin3_w, lin3_b, bn1_g, bn1_b, bn2_g, bn2_b, bn3_g, bn3_b, bn4_g, bn4_b, bn5_g, bn5_b, gat1_w, gat1_asrc, gat1_adst, gat1_bias, gat2_w, gat2_asrc, gat2_adst, gat2_bias):
    raise NotImplementedError("write your pallas kernel here")


# speedup vs baseline: 7.3418x; 7.3418x over previous
import functools

import jax
import jax.numpy as jnp
from jax import lax
from jax.experimental import pallas as pl
from jax.experimental.pallas import tpu as pltpu
from jax.experimental.pallas import tpu_sc as plsc

N = 10000
D_IN = 2613
D = 256
DH = 128
DQ = 64
E = 160000
E_TOT = E + N
NC, NS, L = 2, 16, 16
EPT = 10752
E_PAD = EPT * NS
CH = 128
NCH = EPT // CH
NV = EPT // L
NPAD = 10240
RPT = NPAD // NS
ROW_TILE = 1000
GRID = N // ROW_TILE


def _bn_elu(y, st, gamma, beta):
    m = st[0:1, :] / N
    v = st[1:2, :] / N - m * m
    inv = lax.rsqrt(v + 1e-5)
    h = (y - m) * inv * gamma[None, :] + beta[None, :]
    return jnp.where(h > 0, h, jnp.exp(jnp.minimum(h, 0.0)) - 1.0)


def _mm_stats_body(x_ref, w_ref, b_ref, y_ref, st_ref):
    i = pl.program_id(0)
    y = lax.dot_general(x_ref[...], w_ref[...], (((1,), (1,)), ((), ())),
                        preferred_element_type=jnp.float32)
    y = y + b_ref[...][None, :]
    y_ref[...] = y

    @pl.when(i == 0)
    def _():
        st_ref[...] = jnp.zeros_like(st_ref)

    st_ref[0:1, :] = st_ref[0:1, :] + jnp.sum(y, axis=0, keepdims=True)
    st_ref[1:2, :] = st_ref[1:2, :] + jnp.sum(y * y, axis=0, keepdims=True)


def _lin_stats(x, w, b):
    k = x.shape[1]
    return pl.pallas_call(
        _mm_stats_body,
        grid=(GRID,),
        in_specs=[pl.BlockSpec((ROW_TILE, k), lambda i: (i, 0)),
                  pl.BlockSpec((D, k), lambda i: (0, 0)),
                  pl.BlockSpec((D,), lambda i: (0,))],
        out_specs=[pl.BlockSpec((ROW_TILE, D), lambda i: (i, 0)),
                   pl.BlockSpec((2, D), lambda i: (0, 0))],
        out_shape=[jax.ShapeDtypeStruct((x.shape[0], D), jnp.float32),
                   jax.ShapeDtypeStruct((2, D), jnp.float32)],
    )(x, w, b)


def _bn_lin_body(y_ref, st_ref, gam_ref, bet_ref, w_ref, b_ref, o_ref, sto_ref):
    i = pl.program_id(0)
    h = _bn_elu(y_ref[...], st_ref[...], gam_ref[...], bet_ref[...])
    y2 = lax.dot_general(h, w_ref[...], (((1,), (1,)), ((), ())),
                         preferred_element_type=jnp.float32)
    y2 = y2 + b_ref[...][None, :]
    o_ref[...] = y2

    @pl.when(i == 0)
    def _():
        sto_ref[...] = jnp.zeros_like(sto_ref)

    sto_ref[0:1, :] = sto_ref[0:1, :] + jnp.sum(y2, axis=0, keepdims=True)
    sto_ref[1:2, :] = sto_ref[1:2, :] + jnp.sum(y2 * y2, axis=0, keepdims=True)


def _bn_lin(y, st, gam, bet, w, b):
    return pl.pallas_call(
        _bn_lin_body,
        grid=(GRID,),
        in_specs=[pl.BlockSpec((ROW_TILE, D), lambda i: (i, 0)),
                  pl.BlockSpec((2, D), lambda i: (0, 0)),
                  pl.BlockSpec((D,), lambda i: (0,)),
                  pl.BlockSpec((D,), lambda i: (0,)),
                  pl.BlockSpec((D, D), lambda i: (0, 0)),
                  pl.BlockSpec((D,), lambda i: (0,))],
        out_specs=[pl.BlockSpec((ROW_TILE, D), lambda i: (i, 0)),
                   pl.BlockSpec((2, D), lambda i: (0, 0))],
        out_shape=[jax.ShapeDtypeStruct((N, D), jnp.float32),
                   jax.ShapeDtypeStruct((2, D), jnp.float32)],
    )(y, st, gam, bet, w, b)


def _gat_pre_body(y_ref, st_ref, gam_ref, bet_ref, w_ref, as_ref, ad_ref,
                  g0_ref, g1_ref, g2_ref, g3_ref, s_ref, d_ref):
    h = _bn_elu(y_ref[...], st_ref[...], gam_ref[...], bet_ref[...])
    g = lax.dot_general(h, w_ref[...], (((1,), (1,)), ((), ())),
                        preferred_element_type=jnp.float32)
    g0_ref[...] = g[:, 0 * DQ:1 * DQ]
    g1_ref[...] = g[:, 1 * DQ:2 * DQ]
    g2_ref[...] = g[:, 2 * DQ:3 * DQ]
    g3_ref[...] = g[:, 3 * DQ:4 * DQ]
    s_ref[...] = jnp.sum(g * as_ref[...][None, :], axis=1)[:, None]
    d_ref[...] = jnp.sum(g * ad_ref[...][None, :], axis=1)[:, None]


def _gat_pre(y, st, gam, bet, w, asrc, adst):
    return pl.pallas_call(
        _gat_pre_body,
        grid=(GRID,),
        in_specs=[pl.BlockSpec((ROW_TILE, D), lambda i: (i, 0)),
                  pl.BlockSpec((2, D), lambda i: (0, 0)),
                  pl.BlockSpec((D,), lambda i: (0,)),
                  pl.BlockSpec((D,), lambda i: (0,)),
                  pl.BlockSpec((D, D), lambda i: (0, 0)),
                  pl.BlockSpec((D,), lambda i: (0,)),
                  pl.BlockSpec((D,), lambda i: (0,))],
        out_specs=[pl.BlockSpec((ROW_TILE, DQ), lambda i: (i, 0)),
                   pl.BlockSpec((ROW_TILE, DQ), lambda i: (i, 0)),
                   pl.BlockSpec((ROW_TILE, DQ), lambda i: (i, 0)),
                   pl.BlockSpec((ROW_TILE, DQ), lambda i: (i, 0)),
                   pl.BlockSpec((ROW_TILE, 1), lambda i: (i, 0)),
                   pl.BlockSpec((ROW_TILE, 1), lambda i: (i, 0))],
        out_shape=[jax.ShapeDtypeStruct((N, DQ), jnp.float32),
                   jax.ShapeDtypeStruct((N, DQ), jnp.float32),
                   jax.ShapeDtypeStruct((N, DQ), jnp.float32),
                   jax.ShapeDtypeStruct((N, DQ), jnp.float32),
                   jax.ShapeDtypeStruct((N, 1), jnp.float32),
                   jax.ShapeDtypeStruct((N, 1), jnp.float32)],
    )(y, st, gam, bet, w, asrc, adst)


def _agg_gat_pre_body(a0_ref, a1_ref, a2_ref, a3_ref, stp_ref, gam_ref,
                      bet_ref, w_ref, as_ref, ad_ref,
                      g0_ref, g1_ref, g2_ref, g3_ref, s_ref, d_ref):
    st = jnp.sum(stp_ref[...], axis=1)
    y = jnp.concatenate([a0_ref[0, 0], a1_ref[0, 0], a2_ref[0, 0],
                         a3_ref[0, 0]], axis=1)
    h = _bn_elu(y, st, gam_ref[...], bet_ref[...])
    g = lax.dot_general(h, w_ref[...], (((1,), (1,)), ((), ())),
                        preferred_element_type=jnp.float32)
    g0_ref[...] = g[:, 0 * DQ:1 * DQ]
    g1_ref[...] = g[:, 1 * DQ:2 * DQ]
    g2_ref[...] = g[:, 2 * DQ:3 * DQ]
    g3_ref[...] = g[:, 3 * DQ:4 * DQ]
    s_ref[...] = jnp.sum(g * as_ref[...][None, :], axis=1)[:, None]
    d_ref[...] = jnp.sum(g * ad_ref[...][None, :], axis=1)[:, None]


def _agg_gat_pre(agg, stp, gam, bet, w, asrc, adst):
    return pl.pallas_call(
        _agg_gat_pre_body,
        grid=(GRID,),
        in_specs=[pl.BlockSpec((1, 1, ROW_TILE, DQ), lambda i: (0, 0, i, 0)),
                  pl.BlockSpec((1, 1, ROW_TILE, DQ), lambda i: (0, 1, i, 0)),
                  pl.BlockSpec((1, 1, ROW_TILE, DQ), lambda i: (1, 0, i, 0)),
                  pl.BlockSpec((1, 1, ROW_TILE, DQ), lambda i: (1, 1, i, 0)),
                  pl.BlockSpec((2, NS, D), lambda i: (0, 0, 0)),
                  pl.BlockSpec((D,), lambda i: (0,)),
                  pl.BlockSpec((D,), lambda i: (0,)),
                  pl.BlockSpec((D, D), lambda i: (0, 0)),
                  pl.BlockSpec((D,), lambda i: (0,)),
                  pl.BlockSpec((D,), lambda i: (0,))],
        out_specs=[pl.BlockSpec((ROW_TILE, DQ), lambda i: (i, 0)),
                   pl.BlockSpec((ROW_TILE, DQ), lambda i: (i, 0)),
                   pl.BlockSpec((ROW_TILE, DQ), lambda i: (i, 0)),
                   pl.BlockSpec((ROW_TILE, DQ), lambda i: (i, 0)),
                   pl.BlockSpec((ROW_TILE, 1), lambda i: (i, 0)),
                   pl.BlockSpec((ROW_TILE, 1), lambda i: (i, 0))],
        out_shape=[jax.ShapeDtypeStruct((N, DQ), jnp.float32),
                   jax.ShapeDtypeStruct((N, DQ), jnp.float32),
                   jax.ShapeDtypeStruct((N, DQ), jnp.float32),
                   jax.ShapeDtypeStruct((N, DQ), jnp.float32),
                   jax.ShapeDtypeStruct((N, 1), jnp.float32),
                   jax.ShapeDtypeStruct((N, 1), jnp.float32)],
    )(agg, agg, agg, agg, stp, gam, bet, w, asrc, adst)


def _final_body(a0_ref, a1_ref, a2_ref, a3_ref, stp_ref, gam_ref, bet_ref,
                o_ref):
    st = jnp.sum(stp_ref[...], axis=1)
    y = jnp.concatenate([a0_ref[0, 0], a1_ref[0, 0], a2_ref[0, 0],
                         a3_ref[0, 0]], axis=1)
    o_ref[...] = _bn_elu(y, st, gam_ref[...], bet_ref[...])


def _final(agg, stp, gam, bet):
    return pl.pallas_call(
        _final_body,
        grid=(GRID,),
        in_specs=[pl.BlockSpec((1, 1, ROW_TILE, DQ), lambda i: (0, 0, i, 0)),
                  pl.BlockSpec((1, 1, ROW_TILE, DQ), lambda i: (0, 1, i, 0)),
                  pl.BlockSpec((1, 1, ROW_TILE, DQ), lambda i: (1, 0, i, 0)),
                  pl.BlockSpec((1, 1, ROW_TILE, DQ), lambda i: (1, 1, i, 0)),
                  pl.BlockSpec((2, NS, D), lambda i: (0, 0, 0)),
                  pl.BlockSpec((D,), lambda i: (0,)),
                  pl.BlockSpec((D,), lambda i: (0,))],
        out_specs=pl.BlockSpec((ROW_TILE, D), lambda i: (i, 0)),
        out_shape=jax.ShapeDtypeStruct((N, D), jnp.float32),
    )(agg, agg, agg, agg, stp, gam, bet)


@functools.lru_cache(maxsize=1)
def _make_gat_agg():
  @functools.partial(
    pl.kernel,
    out_type=[jax.ShapeDtypeStruct((NC, 2, NPAD, DQ), jnp.float32),
              jax.ShapeDtypeStruct((NC, 2, NS, 2, DQ), jnp.float32)],
    mesh=plsc.VectorSubcoreMesh(core_axis_name="c", subcore_axis_name="s",
                                num_cores=NC, num_subcores=NS),
    compiler_params=pltpu.CompilerParams(needs_layout_passes=False,
                                         use_tc_tiling_on_sc=False),
    scratch_types=[
        pltpu.VMEM((EPT,), jnp.int32),
        pltpu.VMEM((EPT,), jnp.int32),
        pltpu.VMEM((N,), jnp.float32),
        pltpu.VMEM((N,), jnp.float32),
        pltpu.VMEM((EPT,), jnp.float32),
        pltpu.VMEM((CH, DQ), jnp.float32),
        pltpu.VMEM((CH, DQ), jnp.float32),
        pltpu.VMEM((CH,), jnp.int32),
        pltpu.VMEM((CH,), jnp.float32),
        pltpu.VMEM((RPT,), jnp.float32),
        pltpu.VMEM((L,), jnp.float32),
        pltpu.VMEM((NS, L), jnp.float32),
        pltpu.VMEM((2, DQ), jnp.float32),
        pltpu.VMEM_SHARED((NS, L), jnp.float32),
        pltpu.VMEM_SHARED((NPAD,), jnp.float32),
        pltpu.VMEM_SHARED((NPAD, DQ), jnp.float32),
      ],
  )
  def _gat_agg(src_hbm, dst_hbm, s_hbm, d_hbm, g0_hbm, g1_hbm, g2_hbm, g3_hbm,
               out_hbm, st_hbm,
               src_v, dst_v, s_loc, d_loc, a_v, buf, zbuf, idxw, ew, den_r,
               maxbuf, max_l, statbuf, max_sh, den_sh, acc_sh):
      c = lax.axis_index("c")
      sid = lax.axis_index("s")
      base = sid * EPT
      rb = sid * RPT
      zv = jnp.zeros((L,), jnp.float32)
      NQV = DQ // L

      pltpu.sync_copy(src_hbm.at[pl.ds(base, EPT)], src_v)
      pltpu.sync_copy(dst_hbm.at[pl.ds(base, EPT)], dst_v)
      pltpu.sync_copy(s_hbm, s_loc)
      pltpu.sync_copy(d_hbm, d_loc)

      def alpha_body(i, m):
          si = src_v[pl.ds(i * L, L)]
          di = dst_v[pl.ds(i * L, L)]
          a = plsc.load_gather(s_loc, [si]) + plsc.load_gather(d_loc, [di])
          a = jnp.where(a > 0, a, 0.2 * a)
          eid = base + i * L + lax.iota(jnp.int32, L)
          a = jnp.where(eid < E_TOT, a, -1e30)
          a_v[pl.ds(i * L, L)] = a
          return jnp.maximum(m, a)

      m = lax.fori_loop(0, NV, alpha_body, jnp.full((L,), -1e30, jnp.float32))
      maxbuf[...] = m
      pltpu.sync_copy(maxbuf, max_sh.at[sid])

      def zero_den(i, _):
          den_r[pl.ds(i * L, L)] = zv
          return 0

      lax.fori_loop(0, RPT // L, zero_den, 0)
      pltpu.sync_copy(den_r, den_sh.at[pl.ds(rb, RPT)])

      def zero_zbuf(r, _):
          for v in range(NQV):
              zbuf[r, pl.ds(v * L, L)] = zv
          return 0

      lax.fori_loop(0, CH, zero_zbuf, 0)

      plsc.subcore_barrier()

      pltpu.sync_copy(max_sh, max_l)

      def max_red(i, mm):
          return jnp.maximum(mm, max_l[i])

      m = lax.fori_loop(0, NS, max_red, m)
      gmax = jnp.max(m)

      def den_body(j, _):
          for v in range(CH // L):
              o = j * CH + v * L
              e = jnp.exp(a_v[pl.ds(o, L)] - gmax)
              a_v[pl.ds(o, L)] = e
              ew[pl.ds(v * L, L)] = e
              idxw[pl.ds(v * L, L)] = dst_v[pl.ds(o, L)]
          pltpu.sync_copy(ew, den_sh.at[idxw], add=True)
          return 0

      lax.fori_loop(0, NCH, den_body, 0)
      plsc.subcore_barrier()

      pltpu.sync_copy(den_sh.at[pl.ds(rb, RPT)], den_r)

      def inv_body(i, _):
          dv = den_r[pl.ds(i * L, L)]
          den_r[pl.ds(i * L, L)] = 1.0 / (dv + 1e-16)
          return 0

      lax.fori_loop(0, RPT // L, inv_body, 0)

      def run_pass(g_ref, qi):
          for k in range(RPT // CH):
              pltpu.sync_copy(zbuf, acc_sh.at[pl.ds(rb + k * CH, CH)])
          plsc.subcore_barrier()

          def ch_body(j, _):
              for v in range(CH // L):
                  idxw[pl.ds(v * L, L)] = dst_v[pl.ds(j * CH + v * L, L)]
              pltpu.sync_copy(g_ref.at[src_v.at[pl.ds(j * CH, CH)]], buf)

              def grp_body(gi, _):
                  wv = a_v[pl.ds(j * CH + gi * L, L)]
                  for r16 in range(L):
                      row = gi * L + r16
                      w = wv[r16]
                      for v in range(NQV):
                          sl = pl.ds(v * L, L)
                          buf[row, sl] = buf[row, sl] * w
                  return 0

              lax.fori_loop(0, CH // L, grp_body, 0)
              pltpu.sync_copy(buf, acc_sh.at[idxw], add=True)
              return 0

          lax.fori_loop(0, NCH, ch_body, 0)
          plsc.subcore_barrier()

          ssum = tuple(zv for _ in range(NQV))
          ssq = tuple(zv for _ in range(NQV))
          for k in range(RPT // CH):
              pltpu.sync_copy(acc_sh.at[pl.ds(rb + k * CH, CH)], buf)

              def grp_out(gi, carry):
                  su, sq = list(carry[0]), list(carry[1])
                  iv = den_r[pl.ds(k * CH + gi * L, L)]
                  for r16 in range(L):
                      row = gi * L + r16
                      inv = iv[r16]
                      for v in range(NQV):
                          sl = pl.ds(v * L, L)
                          x = buf[row, sl] * inv
                          buf[row, sl] = x
                          su[v] = su[v] + x
                          sq[v] = sq[v] + x * x
                  return tuple(su), tuple(sq)

              ssum, ssq = lax.fori_loop(0, CH // L, grp_out, (ssum, ssq))
              pltpu.sync_copy(buf, out_hbm.at[c, qi, pl.ds(rb + k * CH, CH)])
          for v in range(NQV):
              statbuf[0, pl.ds(v * L, L)] = ssum[v]
              statbuf[1, pl.ds(v * L, L)] = ssq[v]
          pltpu.sync_copy(statbuf, st_hbm.at[c, qi, sid])
          plsc.subcore_barrier()

      @pl.when(c == 0)
      def _():
          run_pass(g0_hbm, 0)
          run_pass(g1_hbm, 1)

      @pl.when(c == 1)
      def _():
          run_pass(g2_hbm, 0)
          run_pass(g3_hbm, 1)


  return _gat_agg


def kernel(x, edges, lin1_w, lin1_b, lin2_w, lin2_b, lin3_w, lin3_b,
           bn1_g, bn1_b, bn2_g, bn2_b, bn3_g, bn3_b, bn4_g, bn4_b,
           bn5_g, bn5_b,
           gat1_w, gat1_asrc, gat1_adst, gat1_bias,
           gat2_w, gat2_asrc, gat2_adst, gat2_bias):
    del gat1_bias, gat2_bias
    loop = jnp.arange(N, dtype=jnp.int32)
    pad = jnp.arange(E_PAD - E_TOT, dtype=jnp.int32) % N
    src = jnp.concatenate([edges[0], loop, pad])
    dst = jnp.concatenate([edges[1], loop, pad])

    y1, st1 = _lin_stats(x, lin1_w, lin1_b)
    y2, st2 = _bn_lin(y1, st1, bn1_g, bn1_b, lin2_w, lin2_b)
    y3, st3 = _bn_lin(y2, st2, bn2_g, bn2_b, lin3_w, lin3_b)
    g10, g11, g12, g13, s1, d1 = _gat_pre(y3, st3, bn3_g, bn3_b, gat1_w,
                                          gat1_asrc, gat1_adst)
    agg1, stp1 = _make_gat_agg()(src, dst, s1.reshape(N), d1.reshape(N),
                          g10, g11, g12, g13)
    stp1 = jnp.transpose(stp1, (3, 2, 0, 1, 4)).reshape(2, NS, D)
    g20, g21, g22, g23, s2, d2 = _agg_gat_pre(agg1, stp1, bn4_g, bn4_b,
                                              gat2_w, gat2_asrc, gat2_adst)
    agg2, stp2 = _make_gat_agg()(src, dst, s2.reshape(N), d2.reshape(N),
                          g20, g21, g22, g23)
    stp2 = jnp.transpose(stp2, (3, 2, 0, 1, 4)).reshape(2, NS, D)
    return _final(agg2, stp2, bn5_g, bn5_b)


# speedup vs baseline: 10.1295x; 1.3797x over previous
import functools

import jax
import jax.numpy as jnp
from jax import lax
from jax.experimental import pallas as pl
from jax.experimental.pallas import tpu as pltpu
from jax.experimental.pallas import tpu_sc as plsc

N = 10000
D_IN = 2613
D = 256
DH = 128
DQ = 64
E = 160000
E_TOT = E + N
NC, NS, L = 2, 16, 16
EPT = 10752
E_PAD = EPT * NS
CH = 128
NCH = EPT // CH
NV = EPT // L
NPAD = 10240
RPT = NPAD // NS
ROW_TILE = 1000
GRID = N // ROW_TILE


def _bn_elu(y, st, gamma, beta):
    m = st[0:1, :] / N
    v = st[1:2, :] / N - m * m
    inv = lax.rsqrt(v + 1e-5)
    h = (y - m) * inv * gamma[None, :] + beta[None, :]
    return jnp.where(h > 0, h, jnp.exp(jnp.minimum(h, 0.0)) - 1.0)


def _mm_stats_body(x_ref, w_ref, b_ref, y_ref, st_ref):
    i = pl.program_id(0)
    y = lax.dot_general(x_ref[...], w_ref[...], (((1,), (1,)), ((), ())),
                        preferred_element_type=jnp.float32)
    y = y + b_ref[...][None, :]
    y_ref[...] = y

    @pl.when(i == 0)
    def _():
        st_ref[...] = jnp.zeros_like(st_ref)

    st_ref[0:1, :] = st_ref[0:1, :] + jnp.sum(y, axis=0, keepdims=True)
    st_ref[1:2, :] = st_ref[1:2, :] + jnp.sum(y * y, axis=0, keepdims=True)


def _lin_stats(x, w, b):
    k = x.shape[1]
    return pl.pallas_call(
        _mm_stats_body,
        grid=(GRID,),
        in_specs=[pl.BlockSpec((ROW_TILE, k), lambda i: (i, 0)),
                  pl.BlockSpec((D, k), lambda i: (0, 0)),
                  pl.BlockSpec((D,), lambda i: (0,))],
        out_specs=[pl.BlockSpec((ROW_TILE, D), lambda i: (i, 0)),
                   pl.BlockSpec((2, D), lambda i: (0, 0))],
        out_shape=[jax.ShapeDtypeStruct((x.shape[0], D), jnp.float32),
                   jax.ShapeDtypeStruct((2, D), jnp.float32)],
    )(x, w, b)


def _bn_lin_body(y_ref, st_ref, gam_ref, bet_ref, w_ref, b_ref, o_ref, sto_ref):
    i = pl.program_id(0)
    h = _bn_elu(y_ref[...], st_ref[...], gam_ref[...], bet_ref[...])
    y2 = lax.dot_general(h, w_ref[...], (((1,), (1,)), ((), ())),
                         preferred_element_type=jnp.float32)
    y2 = y2 + b_ref[...][None, :]
    o_ref[...] = y2

    @pl.when(i == 0)
    def _():
        sto_ref[...] = jnp.zeros_like(sto_ref)

    sto_ref[0:1, :] = sto_ref[0:1, :] + jnp.sum(y2, axis=0, keepdims=True)
    sto_ref[1:2, :] = sto_ref[1:2, :] + jnp.sum(y2 * y2, axis=0, keepdims=True)


def _bn_lin(y, st, gam, bet, w, b):
    return pl.pallas_call(
        _bn_lin_body,
        grid=(GRID,),
        in_specs=[pl.BlockSpec((ROW_TILE, D), lambda i: (i, 0)),
                  pl.BlockSpec((2, D), lambda i: (0, 0)),
                  pl.BlockSpec((D,), lambda i: (0,)),
                  pl.BlockSpec((D,), lambda i: (0,)),
                  pl.BlockSpec((D, D), lambda i: (0, 0)),
                  pl.BlockSpec((D,), lambda i: (0,))],
        out_specs=[pl.BlockSpec((ROW_TILE, D), lambda i: (i, 0)),
                   pl.BlockSpec((2, D), lambda i: (0, 0))],
        out_shape=[jax.ShapeDtypeStruct((N, D), jnp.float32),
                   jax.ShapeDtypeStruct((2, D), jnp.float32)],
    )(y, st, gam, bet, w, b)


def _gat_pre_body(y_ref, st_ref, gam_ref, bet_ref, w_ref, as_ref, ad_ref,
                  g0_ref, g1_ref, g2_ref, g3_ref, s_ref, d_ref):
    h = _bn_elu(y_ref[...], st_ref[...], gam_ref[...], bet_ref[...])
    g = lax.dot_general(h, w_ref[...], (((1,), (1,)), ((), ())),
                        preferred_element_type=jnp.float32)
    g0_ref[...] = g[:, 0 * DQ:1 * DQ]
    g1_ref[...] = g[:, 1 * DQ:2 * DQ]
    g2_ref[...] = g[:, 2 * DQ:3 * DQ]
    g3_ref[...] = g[:, 3 * DQ:4 * DQ]
    s_ref[...] = jnp.sum(g * as_ref[...][None, :], axis=1)[:, None]
    d_ref[...] = jnp.sum(g * ad_ref[...][None, :], axis=1)[:, None]


def _gat_pre(y, st, gam, bet, w, asrc, adst):
    return pl.pallas_call(
        _gat_pre_body,
        grid=(GRID,),
        in_specs=[pl.BlockSpec((ROW_TILE, D), lambda i: (i, 0)),
                  pl.BlockSpec((2, D), lambda i: (0, 0)),
                  pl.BlockSpec((D,), lambda i: (0,)),
                  pl.BlockSpec((D,), lambda i: (0,)),
                  pl.BlockSpec((D, D), lambda i: (0, 0)),
                  pl.BlockSpec((D,), lambda i: (0,)),
                  pl.BlockSpec((D,), lambda i: (0,))],
        out_specs=[pl.BlockSpec((ROW_TILE, DQ), lambda i: (i, 0)),
                   pl.BlockSpec((ROW_TILE, DQ), lambda i: (i, 0)),
                   pl.BlockSpec((ROW_TILE, DQ), lambda i: (i, 0)),
                   pl.BlockSpec((ROW_TILE, DQ), lambda i: (i, 0)),
                   pl.BlockSpec((ROW_TILE, 1), lambda i: (i, 0)),
                   pl.BlockSpec((ROW_TILE, 1), lambda i: (i, 0))],
        out_shape=[jax.ShapeDtypeStruct((N, DQ), jnp.float32),
                   jax.ShapeDtypeStruct((N, DQ), jnp.float32),
                   jax.ShapeDtypeStruct((N, DQ), jnp.float32),
                   jax.ShapeDtypeStruct((N, DQ), jnp.float32),
                   jax.ShapeDtypeStruct((N, 1), jnp.float32),
                   jax.ShapeDtypeStruct((N, 1), jnp.float32)],
    )(y, st, gam, bet, w, asrc, adst)


def _agg_gat_pre_body(a0_ref, a1_ref, a2_ref, a3_ref, stp_ref, gam_ref,
                      bet_ref, w_ref, as_ref, ad_ref,
                      g0_ref, g1_ref, g2_ref, g3_ref, s_ref, d_ref):
    st = jnp.sum(stp_ref[...], axis=1)
    y = jnp.concatenate([a0_ref[0, 0], a1_ref[0, 0], a2_ref[0, 0],
                         a3_ref[0, 0]], axis=1)
    h = _bn_elu(y, st, gam_ref[...], bet_ref[...])
    g = lax.dot_general(h, w_ref[...], (((1,), (1,)), ((), ())),
                        preferred_element_type=jnp.float32)
    g0_ref[...] = g[:, 0 * DQ:1 * DQ]
    g1_ref[...] = g[:, 1 * DQ:2 * DQ]
    g2_ref[...] = g[:, 2 * DQ:3 * DQ]
    g3_ref[...] = g[:, 3 * DQ:4 * DQ]
    s_ref[...] = jnp.sum(g * as_ref[...][None, :], axis=1)[:, None]
    d_ref[...] = jnp.sum(g * ad_ref[...][None, :], axis=1)[:, None]


def _agg_gat_pre(agg, stp, gam, bet, w, asrc, adst):
    return pl.pallas_call(
        _agg_gat_pre_body,
        grid=(GRID,),
        in_specs=[pl.BlockSpec((1, 1, ROW_TILE, DQ), lambda i: (0, 0, i, 0)),
                  pl.BlockSpec((1, 1, ROW_TILE, DQ), lambda i: (0, 1, i, 0)),
                  pl.BlockSpec((1, 1, ROW_TILE, DQ), lambda i: (1, 0, i, 0)),
                  pl.BlockSpec((1, 1, ROW_TILE, DQ), lambda i: (1, 1, i, 0)),
                  pl.BlockSpec((2, NS, D), lambda i: (0, 0, 0)),
                  pl.BlockSpec((D,), lambda i: (0,)),
                  pl.BlockSpec((D,), lambda i: (0,)),
                  pl.BlockSpec((D, D), lambda i: (0, 0)),
                  pl.BlockSpec((D,), lambda i: (0,)),
                  pl.BlockSpec((D,), lambda i: (0,))],
        out_specs=[pl.BlockSpec((ROW_TILE, DQ), lambda i: (i, 0)),
                   pl.BlockSpec((ROW_TILE, DQ), lambda i: (i, 0)),
                   pl.BlockSpec((ROW_TILE, DQ), lambda i: (i, 0)),
                   pl.BlockSpec((ROW_TILE, DQ), lambda i: (i, 0)),
                   pl.BlockSpec((ROW_TILE, 1), lambda i: (i, 0)),
                   pl.BlockSpec((ROW_TILE, 1), lambda i: (i, 0))],
        out_shape=[jax.ShapeDtypeStruct((N, DQ), jnp.float32),
                   jax.ShapeDtypeStruct((N, DQ), jnp.float32),
                   jax.ShapeDtypeStruct((N, DQ), jnp.float32),
                   jax.ShapeDtypeStruct((N, DQ), jnp.float32),
                   jax.ShapeDtypeStruct((N, 1), jnp.float32),
                   jax.ShapeDtypeStruct((N, 1), jnp.float32)],
    )(agg, agg, agg, agg, stp, gam, bet, w, asrc, adst)


def _final_body(a0_ref, a1_ref, a2_ref, a3_ref, stp_ref, gam_ref, bet_ref,
                o_ref):
    st = jnp.sum(stp_ref[...], axis=1)
    y = jnp.concatenate([a0_ref[0, 0], a1_ref[0, 0], a2_ref[0, 0],
                         a3_ref[0, 0]], axis=1)
    o_ref[...] = _bn_elu(y, st, gam_ref[...], bet_ref[...])


def _final(agg, stp, gam, bet):
    return pl.pallas_call(
        _final_body,
        grid=(GRID,),
        in_specs=[pl.BlockSpec((1, 1, ROW_TILE, DQ), lambda i: (0, 0, i, 0)),
                  pl.BlockSpec((1, 1, ROW_TILE, DQ), lambda i: (0, 1, i, 0)),
                  pl.BlockSpec((1, 1, ROW_TILE, DQ), lambda i: (1, 0, i, 0)),
                  pl.BlockSpec((1, 1, ROW_TILE, DQ), lambda i: (1, 1, i, 0)),
                  pl.BlockSpec((2, NS, D), lambda i: (0, 0, 0)),
                  pl.BlockSpec((D,), lambda i: (0,)),
                  pl.BlockSpec((D,), lambda i: (0,))],
        out_specs=pl.BlockSpec((ROW_TILE, D), lambda i: (i, 0)),
        out_shape=jax.ShapeDtypeStruct((N, D), jnp.float32),
    )(agg, agg, agg, agg, stp, gam, bet)


@functools.lru_cache(maxsize=1)
def _make_gat_agg():
  @functools.partial(
    pl.kernel,
    out_type=[jax.ShapeDtypeStruct((NC, 2, NPAD, DQ), jnp.float32),
              jax.ShapeDtypeStruct((NC, 2, NS, 2, DQ), jnp.float32)],
    mesh=plsc.VectorSubcoreMesh(core_axis_name="c", subcore_axis_name="s",
                                num_cores=NC, num_subcores=NS),
    compiler_params=pltpu.CompilerParams(needs_layout_passes=False,
                                         use_tc_tiling_on_sc=False),
    scratch_types=[
        pltpu.VMEM((EPT,), jnp.int32),
        pltpu.VMEM((EPT,), jnp.int32),
        pltpu.VMEM((N,), jnp.float32),
        pltpu.VMEM((N,), jnp.float32),
        pltpu.VMEM((EPT,), jnp.float32),
        pltpu.VMEM((CH, DQ), jnp.float32),
        pltpu.VMEM((CH, DQ), jnp.float32),
        pltpu.VMEM((CH, DQ), jnp.float32),
        pltpu.VMEM((CH,), jnp.int32),
        pltpu.VMEM((CH,), jnp.int32),
        pltpu.VMEM((CH,), jnp.float32),
        pltpu.VMEM((CH,), jnp.float32),
        pltpu.SemaphoreType.DMA,
        pltpu.SemaphoreType.DMA,
        pltpu.SemaphoreType.DMA,
        pltpu.SemaphoreType.DMA,
        pltpu.VMEM((RPT,), jnp.float32),
        pltpu.VMEM((L,), jnp.float32),
        pltpu.VMEM((NS, L), jnp.float32),
        pltpu.VMEM((2, DQ), jnp.float32),
        pltpu.VMEM_SHARED((NS, L), jnp.float32),
        pltpu.VMEM_SHARED((NPAD,), jnp.float32),
        pltpu.VMEM_SHARED((NPAD, DQ), jnp.float32),
      ],
  )
  def _gat_agg(src_hbm, dst_hbm, s_hbm, d_hbm, g0_hbm, g1_hbm, g2_hbm, g3_hbm,
               out_hbm, st_hbm,
               src_v, dst_v, s_loc, d_loc, a_v, buf, buf2, zbuf, idxw,
               idxw2, ew, ew2, gsem0, gsem1, ssem0, ssem1, den_r,
               maxbuf, max_l, statbuf, max_sh, den_sh, acc_sh):
      c = lax.axis_index("c")
      sid = lax.axis_index("s")
      base = sid * EPT
      rb = sid * RPT
      zv = jnp.zeros((L,), jnp.float32)
      NQV = DQ // L

      pltpu.sync_copy(src_hbm.at[pl.ds(base, EPT)], src_v)
      pltpu.sync_copy(dst_hbm.at[pl.ds(base, EPT)], dst_v)
      pltpu.sync_copy(s_hbm, s_loc)
      pltpu.sync_copy(d_hbm, d_loc)

      def alpha_body(i, m):
          si = src_v[pl.ds(i * L, L)]
          di = dst_v[pl.ds(i * L, L)]
          a = plsc.load_gather(s_loc, [si]) + plsc.load_gather(d_loc, [di])
          a = jnp.where(a > 0, a, 0.2 * a)
          eid = base + i * L + lax.iota(jnp.int32, L)
          a = jnp.where(eid < E_TOT, a, -1e30)
          a_v[pl.ds(i * L, L)] = a
          return jnp.maximum(m, a)

      m = lax.fori_loop(0, NV, alpha_body, jnp.full((L,), -1e30, jnp.float32))
      maxbuf[...] = m
      pltpu.sync_copy(maxbuf, max_sh.at[sid])

      def zero_den(i, _):
          den_r[pl.ds(i * L, L)] = zv
          return 0

      lax.fori_loop(0, RPT // L, zero_den, 0)
      pltpu.sync_copy(den_r, den_sh.at[pl.ds(rb, RPT)])

      def zero_zbuf(r, _):
          for v in range(NQV):
              zbuf[r, pl.ds(v * L, L)] = zv
          return 0

      lax.fori_loop(0, CH, zero_zbuf, 0)

      plsc.subcore_barrier()

      pltpu.sync_copy(max_sh, max_l)

      def max_red(i, mm):
          return jnp.maximum(mm, max_l[i])

      m = lax.fori_loop(0, NS, max_red, m)
      gmax = jnp.max(m)

      ews = (ew, ew2)
      idxws_ = (idxw, idxw2)
      ssems_ = (ssem0, ssem1)

      def den_body(j2, _):
          for p in range(2):
              j = j2 * 2 + p
              eb, ib, ss = ews[p], idxws_[p], ssems_[p]

              @pl.when(j2 > 0)
              def _():
                  pltpu.make_async_copy(eb, den_sh.at[ib], ss).wait()

              for v in range(CH // L):
                  o = j * CH + v * L
                  e = jnp.exp(a_v[pl.ds(o, L)] - gmax)
                  a_v[pl.ds(o, L)] = e
                  eb[pl.ds(v * L, L)] = e
                  ib[pl.ds(v * L, L)] = dst_v[pl.ds(o, L)]
              pltpu.async_copy(eb, den_sh.at[ib], ss, add=True)
          return 0

      lax.fori_loop(0, NCH // 2, den_body, 0)
      for p in range(2):
          pltpu.make_async_copy(ews[p], den_sh.at[idxws_[p]], ssems_[p]).wait()
      plsc.subcore_barrier()

      pltpu.sync_copy(den_sh.at[pl.ds(rb, RPT)], den_r)

      def inv_body(i, _):
          dv = den_r[pl.ds(i * L, L)]
          den_r[pl.ds(i * L, L)] = 1.0 / (dv + 1e-16)
          return 0

      lax.fori_loop(0, RPT // L, inv_body, 0)

      bufs = (buf, buf2)
      idxws = (idxw, idxw2)
      gsems = (gsem0, gsem1)
      ssems = (ssem0, ssem1)

      def run_pass(g_ref, qi):
          for k in range(RPT // CH):
              pltpu.sync_copy(zbuf, acc_sh.at[pl.ds(rb + k * CH, CH)])
          plsc.subcore_barrier()

          pltpu.async_copy(g_ref.at[src_v.at[pl.ds(0, CH)]], buf, gsem0)

          def ch2_body(j2, _):
              for p in range(2):
                  j = j2 * 2 + p
                  b, ib, gs, ss = bufs[p], idxws[p], gsems[p], ssems[p]

                  @pl.when(j2 > 0)
                  def _():
                      pltpu.make_async_copy(b, acc_sh.at[ib], ss).wait()

                  @pl.when(j + 1 < NCH)
                  def _():
                      pltpu.async_copy(
                          g_ref.at[src_v.at[pl.ds((j + 1) * CH, CH)]],
                          bufs[1 - p], gsems[1 - p])

                  pltpu.make_async_copy(
                      g_ref.at[src_v.at[pl.ds(j * CH, CH)]], b, gs).wait()
                  for v in range(CH // L):
                      ib[pl.ds(v * L, L)] = dst_v[pl.ds(j * CH + v * L, L)]

                  def grp_body(gi, _):
                      wv = a_v[pl.ds(j * CH + gi * L, L)]
                      for r16 in range(L):
                          row = gi * L + r16
                          w = wv[r16]
                          for v in range(NQV):
                              sl = pl.ds(v * L, L)
                              b[row, sl] = b[row, sl] * w
                      return 0

                  lax.fori_loop(0, CH // L, grp_body, 0)
                  pltpu.async_copy(b, acc_sh.at[ib], ss, add=True)
              return 0

          lax.fori_loop(0, NCH // 2, ch2_body, 0)
          for p in range(2):
              pltpu.make_async_copy(bufs[p], acc_sh.at[idxws[p]],
                                    ssems[p]).wait()
          plsc.subcore_barrier()

          ssum = tuple(zv for _ in range(NQV))
          ssq = tuple(zv for _ in range(NQV))
          for k in range(RPT // CH):
              pltpu.sync_copy(acc_sh.at[pl.ds(rb + k * CH, CH)], buf)

              def grp_out(gi, carry):
                  su, sq = list(carry[0]), list(carry[1])
                  iv = den_r[pl.ds(k * CH + gi * L, L)]
                  for r16 in range(L):
                      row = gi * L + r16
                      inv = iv[r16]
                      for v in range(NQV):
                          sl = pl.ds(v * L, L)
                          x = buf[row, sl] * inv
                          buf[row, sl] = x
                          su[v] = su[v] + x
                          sq[v] = sq[v] + x * x
                  return tuple(su), tuple(sq)

              ssum, ssq = lax.fori_loop(0, CH // L, grp_out, (ssum, ssq))
              pltpu.sync_copy(buf, out_hbm.at[c, qi, pl.ds(rb + k * CH, CH)])
          for v in range(NQV):
              statbuf[0, pl.ds(v * L, L)] = ssum[v]
              statbuf[1, pl.ds(v * L, L)] = ssq[v]
          pltpu.sync_copy(statbuf, st_hbm.at[c, qi, sid])
          plsc.subcore_barrier()

      @pl.when(c == 0)
      def _():
          run_pass(g0_hbm, 0)
          run_pass(g1_hbm, 1)

      @pl.when(c == 1)
      def _():
          run_pass(g2_hbm, 0)
          run_pass(g3_hbm, 1)


  return _gat_agg


def kernel(x, edges, lin1_w, lin1_b, lin2_w, lin2_b, lin3_w, lin3_b,
           bn1_g, bn1_b, bn2_g, bn2_b, bn3_g, bn3_b, bn4_g, bn4_b,
           bn5_g, bn5_b,
           gat1_w, gat1_asrc, gat1_adst, gat1_bias,
           gat2_w, gat2_asrc, gat2_adst, gat2_bias):
    del gat1_bias, gat2_bias
    loop = jnp.arange(N, dtype=jnp.int32)
    pad = jnp.arange(E_PAD - E_TOT, dtype=jnp.int32) % N
    src = jnp.concatenate([edges[0], loop, pad])
    dst = jnp.concatenate([edges[1], loop, pad])

    y1, st1 = _lin_stats(x, lin1_w, lin1_b)
    y2, st2 = _bn_lin(y1, st1, bn1_g, bn1_b, lin2_w, lin2_b)
    y3, st3 = _bn_lin(y2, st2, bn2_g, bn2_b, lin3_w, lin3_b)
    g10, g11, g12, g13, s1, d1 = _gat_pre(y3, st3, bn3_g, bn3_b, gat1_w,
                                          gat1_asrc, gat1_adst)
    agg1, stp1 = _make_gat_agg()(src, dst, s1.reshape(N), d1.reshape(N),
                          g10, g11, g12, g13)
    stp1 = jnp.transpose(stp1, (3, 2, 0, 1, 4)).reshape(2, NS, D)
    g20, g21, g22, g23, s2, d2 = _agg_gat_pre(agg1, stp1, bn4_g, bn4_b,
                                              gat2_w, gat2_asrc, gat2_adst)
    agg2, stp2 = _make_gat_agg()(src, dst, s2.reshape(N), d2.reshape(N),
                          g20, g21, g22, g23)
    stp2 = jnp.transpose(stp2, (3, 2, 0, 1, 4)).reshape(2, NS, D)
    return _final(agg2, stp2, bn5_g, bn5_b)


# speedup vs baseline: 16.6752x; 1.6462x over previous
import functools

import jax
import jax.numpy as jnp
from jax import lax
from jax.experimental import pallas as pl
from jax.experimental.pallas import tpu as pltpu
from jax.experimental.pallas import tpu_sc as plsc

N = 10000
D_IN = 2613
D = 256
DH = 128
DQ = 64
E = 160000
E_TOT = E + N
NC, NS, L = 2, 16, 16
EPT = 10752
E_PAD = EPT * NS
CH = 128
NCH = EPT // CH
NV = EPT // L
NPAD = 10240
RPT = NPAD // NS
ROW_TILE = 1000
GRID = N // ROW_TILE


def _bn_elu(y, st, gamma, beta):
    m = st[0:1, :] / N
    v = st[1:2, :] / N - m * m
    inv = lax.rsqrt(v + 1e-5)
    h = (y - m) * inv * gamma[None, :] + beta[None, :]
    return jnp.where(h > 0, h, jnp.exp(jnp.minimum(h, 0.0)) - 1.0)


def _mm_stats_body(x_ref, w_ref, b_ref, y_ref, st_ref):
    i = pl.program_id(0)
    y = lax.dot_general(x_ref[...], w_ref[...], (((1,), (1,)), ((), ())),
                        preferred_element_type=jnp.float32)
    y = y + b_ref[...][None, :]
    y_ref[...] = y

    @pl.when(i == 0)
    def _():
        st_ref[...] = jnp.zeros_like(st_ref)

    st_ref[0:1, :] = st_ref[0:1, :] + jnp.sum(y, axis=0, keepdims=True)
    st_ref[1:2, :] = st_ref[1:2, :] + jnp.sum(y * y, axis=0, keepdims=True)


def _lin_stats(x, w, b):
    k = x.shape[1]
    return pl.pallas_call(
        _mm_stats_body,
        grid=(GRID,),
        in_specs=[pl.BlockSpec((ROW_TILE, k), lambda i: (i, 0)),
                  pl.BlockSpec((D, k), lambda i: (0, 0)),
                  pl.BlockSpec((D,), lambda i: (0,))],
        out_specs=[pl.BlockSpec((ROW_TILE, D), lambda i: (i, 0)),
                   pl.BlockSpec((2, D), lambda i: (0, 0))],
        out_shape=[jax.ShapeDtypeStruct((x.shape[0], D), jnp.float32),
                   jax.ShapeDtypeStruct((2, D), jnp.float32)],
    )(x, w, b)


def _bn_lin_body(y_ref, st_ref, gam_ref, bet_ref, w_ref, b_ref, o_ref, sto_ref):
    i = pl.program_id(0)
    h = _bn_elu(y_ref[...], st_ref[...], gam_ref[...], bet_ref[...])
    y2 = lax.dot_general(h, w_ref[...], (((1,), (1,)), ((), ())),
                         preferred_element_type=jnp.float32)
    y2 = y2 + b_ref[...][None, :]
    o_ref[...] = y2

    @pl.when(i == 0)
    def _():
        sto_ref[...] = jnp.zeros_like(sto_ref)

    sto_ref[0:1, :] = sto_ref[0:1, :] + jnp.sum(y2, axis=0, keepdims=True)
    sto_ref[1:2, :] = sto_ref[1:2, :] + jnp.sum(y2 * y2, axis=0, keepdims=True)


def _bn_lin(y, st, gam, bet, w, b):
    return pl.pallas_call(
        _bn_lin_body,
        grid=(GRID,),
        in_specs=[pl.BlockSpec((ROW_TILE, D), lambda i: (i, 0)),
                  pl.BlockSpec((2, D), lambda i: (0, 0)),
                  pl.BlockSpec((D,), lambda i: (0,)),
                  pl.BlockSpec((D,), lambda i: (0,)),
                  pl.BlockSpec((D, D), lambda i: (0, 0)),
                  pl.BlockSpec((D,), lambda i: (0,))],
        out_specs=[pl.BlockSpec((ROW_TILE, D), lambda i: (i, 0)),
                   pl.BlockSpec((2, D), lambda i: (0, 0))],
        out_shape=[jax.ShapeDtypeStruct((N, D), jnp.float32),
                   jax.ShapeDtypeStruct((2, D), jnp.float32)],
    )(y, st, gam, bet, w, b)


def _gat_pre_body(y_ref, st_ref, gam_ref, bet_ref, w_ref, as_ref, ad_ref,
                  g0_ref, g1_ref, g2_ref, g3_ref, s_ref, d_ref):
    h = _bn_elu(y_ref[...], st_ref[...], gam_ref[...], bet_ref[...])
    g = lax.dot_general(h, w_ref[...], (((1,), (1,)), ((), ())),
                        preferred_element_type=jnp.float32)
    g0_ref[...] = g[:, 0 * DQ:1 * DQ]
    g1_ref[...] = g[:, 1 * DQ:2 * DQ]
    g2_ref[...] = g[:, 2 * DQ:3 * DQ]
    g3_ref[...] = g[:, 3 * DQ:4 * DQ]
    s_ref[...] = jnp.sum(g * as_ref[...][None, :], axis=1)[:, None]
    d_ref[...] = jnp.sum(g * ad_ref[...][None, :], axis=1)[:, None]


def _gat_pre(y, st, gam, bet, w, asrc, adst):
    return pl.pallas_call(
        _gat_pre_body,
        grid=(GRID,),
        in_specs=[pl.BlockSpec((ROW_TILE, D), lambda i: (i, 0)),
                  pl.BlockSpec((2, D), lambda i: (0, 0)),
                  pl.BlockSpec((D,), lambda i: (0,)),
                  pl.BlockSpec((D,), lambda i: (0,)),
                  pl.BlockSpec((D, D), lambda i: (0, 0)),
                  pl.BlockSpec((D,), lambda i: (0,)),
                  pl.BlockSpec((D,), lambda i: (0,))],
        out_specs=[pl.BlockSpec((ROW_TILE, DQ), lambda i: (i, 0)),
                   pl.BlockSpec((ROW_TILE, DQ), lambda i: (i, 0)),
                   pl.BlockSpec((ROW_TILE, DQ), lambda i: (i, 0)),
                   pl.BlockSpec((ROW_TILE, DQ), lambda i: (i, 0)),
                   pl.BlockSpec((ROW_TILE, 1), lambda i: (i, 0)),
                   pl.BlockSpec((ROW_TILE, 1), lambda i: (i, 0))],
        out_shape=[jax.ShapeDtypeStruct((N, DQ), jnp.float32),
                   jax.ShapeDtypeStruct((N, DQ), jnp.float32),
                   jax.ShapeDtypeStruct((N, DQ), jnp.float32),
                   jax.ShapeDtypeStruct((N, DQ), jnp.float32),
                   jax.ShapeDtypeStruct((N, 1), jnp.float32),
                   jax.ShapeDtypeStruct((N, 1), jnp.float32)],
    )(y, st, gam, bet, w, asrc, adst)


def _agg_gat_pre_body(a0_ref, a1_ref, a2_ref, a3_ref, stp_ref, gam_ref,
                      bet_ref, w_ref, as_ref, ad_ref,
                      g0_ref, g1_ref, g2_ref, g3_ref, s_ref, d_ref):
    st = jnp.sum(stp_ref[...], axis=1)
    y = jnp.concatenate([a0_ref[0, 0], a1_ref[0, 0], a2_ref[0, 0],
                         a3_ref[0, 0]], axis=1)
    h = _bn_elu(y, st, gam_ref[...], bet_ref[...])
    g = lax.dot_general(h, w_ref[...], (((1,), (1,)), ((), ())),
                        preferred_element_type=jnp.float32)
    g0_ref[...] = g[:, 0 * DQ:1 * DQ]
    g1_ref[...] = g[:, 1 * DQ:2 * DQ]
    g2_ref[...] = g[:, 2 * DQ:3 * DQ]
    g3_ref[...] = g[:, 3 * DQ:4 * DQ]
    s_ref[...] = jnp.sum(g * as_ref[...][None, :], axis=1)[:, None]
    d_ref[...] = jnp.sum(g * ad_ref[...][None, :], axis=1)[:, None]


def _agg_gat_pre(agg, stp, gam, bet, w, asrc, adst):
    return pl.pallas_call(
        _agg_gat_pre_body,
        grid=(GRID,),
        in_specs=[pl.BlockSpec((1, 1, ROW_TILE, DQ), lambda i: (0, 0, i, 0)),
                  pl.BlockSpec((1, 1, ROW_TILE, DQ), lambda i: (0, 1, i, 0)),
                  pl.BlockSpec((1, 1, ROW_TILE, DQ), lambda i: (1, 0, i, 0)),
                  pl.BlockSpec((1, 1, ROW_TILE, DQ), lambda i: (1, 1, i, 0)),
                  pl.BlockSpec((2, NS, D), lambda i: (0, 0, 0)),
                  pl.BlockSpec((D,), lambda i: (0,)),
                  pl.BlockSpec((D,), lambda i: (0,)),
                  pl.BlockSpec((D, D), lambda i: (0, 0)),
                  pl.BlockSpec((D,), lambda i: (0,)),
                  pl.BlockSpec((D,), lambda i: (0,))],
        out_specs=[pl.BlockSpec((ROW_TILE, DQ), lambda i: (i, 0)),
                   pl.BlockSpec((ROW_TILE, DQ), lambda i: (i, 0)),
                   pl.BlockSpec((ROW_TILE, DQ), lambda i: (i, 0)),
                   pl.BlockSpec((ROW_TILE, DQ), lambda i: (i, 0)),
                   pl.BlockSpec((ROW_TILE, 1), lambda i: (i, 0)),
                   pl.BlockSpec((ROW_TILE, 1), lambda i: (i, 0))],
        out_shape=[jax.ShapeDtypeStruct((N, DQ), jnp.float32),
                   jax.ShapeDtypeStruct((N, DQ), jnp.float32),
                   jax.ShapeDtypeStruct((N, DQ), jnp.float32),
                   jax.ShapeDtypeStruct((N, DQ), jnp.float32),
                   jax.ShapeDtypeStruct((N, 1), jnp.float32),
                   jax.ShapeDtypeStruct((N, 1), jnp.float32)],
    )(agg, agg, agg, agg, stp, gam, bet, w, asrc, adst)


def _final_body(a0_ref, a1_ref, a2_ref, a3_ref, stp_ref, gam_ref, bet_ref,
                o_ref):
    st = jnp.sum(stp_ref[...], axis=1)
    y = jnp.concatenate([a0_ref[0, 0], a1_ref[0, 0], a2_ref[0, 0],
                         a3_ref[0, 0]], axis=1)
    o_ref[...] = _bn_elu(y, st, gam_ref[...], bet_ref[...])


def _final(agg, stp, gam, bet):
    return pl.pallas_call(
        _final_body,
        grid=(GRID,),
        in_specs=[pl.BlockSpec((1, 1, ROW_TILE, DQ), lambda i: (0, 0, i, 0)),
                  pl.BlockSpec((1, 1, ROW_TILE, DQ), lambda i: (0, 1, i, 0)),
                  pl.BlockSpec((1, 1, ROW_TILE, DQ), lambda i: (1, 0, i, 0)),
                  pl.BlockSpec((1, 1, ROW_TILE, DQ), lambda i: (1, 1, i, 0)),
                  pl.BlockSpec((2, NS, D), lambda i: (0, 0, 0)),
                  pl.BlockSpec((D,), lambda i: (0,)),
                  pl.BlockSpec((D,), lambda i: (0,))],
        out_specs=pl.BlockSpec((ROW_TILE, D), lambda i: (i, 0)),
        out_shape=jax.ShapeDtypeStruct((N, D), jnp.float32),
    )(agg, agg, agg, agg, stp, gam, bet)


@functools.lru_cache(maxsize=1)
def _make_gat_agg():
  @functools.partial(
    pl.kernel,
    out_type=[jax.ShapeDtypeStruct((NC, 2, NPAD, DQ), jnp.float32),
              jax.ShapeDtypeStruct((NC, 2, NS, 2, DQ), jnp.float32)],
    mesh=plsc.VectorSubcoreMesh(core_axis_name="c", subcore_axis_name="s",
                                num_cores=NC, num_subcores=NS),
    compiler_params=pltpu.CompilerParams(needs_layout_passes=False,
                                         use_tc_tiling_on_sc=False),
    scratch_types=[
        pltpu.VMEM((EPT,), jnp.int32),
        pltpu.VMEM((EPT,), jnp.int32),
        pltpu.VMEM((N,), jnp.float32),
        pltpu.VMEM((N,), jnp.float32),
        pltpu.VMEM((EPT,), jnp.float32),
        pltpu.VMEM((CH, DQ), jnp.float32),
        pltpu.VMEM((CH, DQ), jnp.float32),
        pltpu.VMEM((CH, DQ), jnp.float32),
        pltpu.VMEM((CH, DQ), jnp.float32),
        pltpu.VMEM((CH,), jnp.int32),
        pltpu.VMEM((CH,), jnp.int32),
        pltpu.VMEM((CH,), jnp.float32),
        pltpu.VMEM((CH,), jnp.float32),
        pltpu.SemaphoreType.DMA,
        pltpu.SemaphoreType.DMA,
        pltpu.SemaphoreType.DMA,
        pltpu.SemaphoreType.DMA,
        pltpu.VMEM((RPT,), jnp.float32),
        pltpu.VMEM((L,), jnp.float32),
        pltpu.VMEM((NS, L), jnp.float32),
        pltpu.VMEM((2, DQ), jnp.float32),
        pltpu.VMEM_SHARED((NS, L), jnp.float32),
        pltpu.VMEM_SHARED((NPAD,), jnp.float32),
        pltpu.VMEM_SHARED((NPAD, DQ), jnp.float32),
      ],
  )
  def _gat_agg(src_hbm, dst_hbm, s_hbm, d_hbm, g0_hbm, g1_hbm, g2_hbm, g3_hbm,
               out_hbm, st_hbm,
               src_v, dst_v, s_loc, d_loc, a_v, buf, buf2, sbuf, sbuf2,
               idxw, idxw2, ew, ew2, gsem0, gsem1, ssem0, ssem1, den_r,
               maxbuf, max_l, statbuf, max_sh, den_sh, acc_sh):
      c = lax.axis_index("c")
      sid = lax.axis_index("s")
      base = sid * EPT
      rb = sid * RPT
      zv = jnp.zeros((L,), jnp.float32)
      NQV = DQ // L

      pltpu.sync_copy(src_hbm.at[pl.ds(base, EPT)], src_v)
      pltpu.sync_copy(dst_hbm.at[pl.ds(base, EPT)], dst_v)
      pltpu.sync_copy(s_hbm, s_loc)
      pltpu.sync_copy(d_hbm, d_loc)

      def alpha_body(i, m):
          si = src_v[pl.ds(i * L, L)]
          di = dst_v[pl.ds(i * L, L)]
          a = plsc.load_gather(s_loc, [si]) + plsc.load_gather(d_loc, [di])
          a = jnp.where(a > 0, a, 0.2 * a)
          eid = base + i * L + lax.iota(jnp.int32, L)
          a = jnp.where(eid < E_TOT, a, -1e30)
          a_v[pl.ds(i * L, L)] = a
          return jnp.maximum(m, a)

      m = lax.fori_loop(0, NV, alpha_body, jnp.full((L,), -1e30, jnp.float32))
      maxbuf[...] = m
      pltpu.sync_copy(maxbuf, max_sh.at[sid])

      def zero_den(i, _):
          den_r[pl.ds(i * L, L)] = zv
          return 0

      lax.fori_loop(0, RPT // L, zero_den, 0)
      pltpu.sync_copy(den_r, den_sh.at[pl.ds(rb, RPT)])


      plsc.subcore_barrier()

      pltpu.sync_copy(max_sh, max_l)

      def max_red(i, mm):
          return jnp.maximum(mm, max_l[i])

      m = lax.fori_loop(0, NS, max_red, m)
      gmax = jnp.max(m)

      ews = (ew, ew2)
      idxws_ = (idxw, idxw2)
      ssems_ = (ssem0, ssem1)

      def den_body(j2, _):
          for p in range(2):
              j = j2 * 2 + p
              eb, ib, ss = ews[p], idxws_[p], ssems_[p]

              @pl.when(j2 > 0)
              def _():
                  pltpu.make_async_copy(eb, den_sh.at[ib], ss).wait()

              for v in range(CH // L):
                  o = j * CH + v * L
                  e = jnp.exp(a_v[pl.ds(o, L)] - gmax)
                  a_v[pl.ds(o, L)] = e
                  eb[pl.ds(v * L, L)] = e
                  ib[pl.ds(v * L, L)] = dst_v[pl.ds(o, L)]
              pltpu.async_copy(eb, den_sh.at[ib], ss, add=True)
          return 0

      lax.fori_loop(0, NCH // 2, den_body, 0)
      for p in range(2):
          pltpu.make_async_copy(ews[p], den_sh.at[idxws_[p]], ssems_[p]).wait()
      plsc.subcore_barrier()

      pltpu.sync_copy(den_sh.at[pl.ds(rb, RPT)], den_r)

      def inv_body(i, _):
          dv = den_r[pl.ds(i * L, L)]
          den_r[pl.ds(i * L, L)] = 1.0 / (dv + 1e-16)
          return 0

      lax.fori_loop(0, RPT // L, inv_body, 0)

      bufs = (buf, buf2)
      sbufs = (sbuf, sbuf2)
      idxws = (idxw, idxw2)
      gsems = (gsem0, gsem1)
      ssems = (ssem0, ssem1)

      def zero_sbuf(r, _):
          for v in range(NQV):
              sbuf[r, pl.ds(v * L, L)] = zv
          return 0

      def run_pass(g_ref, qi):
          lax.fori_loop(0, CH, zero_sbuf, 0)
          for k in range(RPT // CH):
              pltpu.sync_copy(sbuf, acc_sh.at[pl.ds(rb + k * CH, CH)])
          plsc.subcore_barrier()

          pltpu.async_copy(g_ref.at[src_v.at[pl.ds(0, CH)]], buf, gsem0)

          def ch2_body(j2, _):
              for p in range(2):
                  j = j2 * 2 + p
                  b, sb, ib = bufs[p], sbufs[p], idxws[p]
                  gs, ss = gsems[p], ssems[p]

                  @pl.when(j2 > 0)
                  def _():
                      pltpu.make_async_copy(sb, acc_sh.at[ib], ss).wait()

                  @pl.when(j + 1 < NCH)
                  def _():
                      pltpu.async_copy(
                          g_ref.at[src_v.at[pl.ds((j + 1) * CH, CH)]],
                          bufs[1 - p], gsems[1 - p])

                  pltpu.make_async_copy(
                      g_ref.at[src_v.at[pl.ds(j * CH, CH)]], b, gs).wait()
                  for v in range(CH // L):
                      ib[pl.ds(v * L, L)] = dst_v[pl.ds(j * CH + v * L, L)]

                  def grp_body(gi, _):
                      wv = a_v[pl.ds(j * CH + gi * L, L)]
                      for r16 in range(L):
                          row = gi * L + r16
                          w = wv[r16]
                          for v in range(NQV):
                              sl = pl.ds(v * L, L)
                              sb[row, sl] = b[row, sl] * w
                      return 0

                  lax.fori_loop(0, CH // L, grp_body, 0)
                  pltpu.async_copy(sb, acc_sh.at[ib], ss, add=True)
              return 0

          lax.fori_loop(0, NCH // 2, ch2_body, 0)
          for p in range(2):
              pltpu.make_async_copy(sbufs[p], acc_sh.at[idxws[p]],
                                    ssems[p]).wait()
          plsc.subcore_barrier()

          ssum = tuple(zv for _ in range(NQV))
          ssq = tuple(zv for _ in range(NQV))
          for k in range(RPT // CH):
              pltpu.sync_copy(acc_sh.at[pl.ds(rb + k * CH, CH)], buf)

              def grp_out(gi, carry):
                  su, sq = list(carry[0]), list(carry[1])
                  iv = den_r[pl.ds(k * CH + gi * L, L)]
                  for r16 in range(L):
                      row = gi * L + r16
                      inv = iv[r16]
                      for v in range(NQV):
                          sl = pl.ds(v * L, L)
                          x = buf[row, sl] * inv
                          buf[row, sl] = x
                          su[v] = su[v] + x
                          sq[v] = sq[v] + x * x
                  return tuple(su), tuple(sq)

              ssum, ssq = lax.fori_loop(0, CH // L, grp_out, (ssum, ssq))
              pltpu.sync_copy(buf, out_hbm.at[c, qi, pl.ds(rb + k * CH, CH)])
          for v in range(NQV):
              statbuf[0, pl.ds(v * L, L)] = ssum[v]
              statbuf[1, pl.ds(v * L, L)] = ssq[v]
          pltpu.sync_copy(statbuf, st_hbm.at[c, qi, sid])
          plsc.subcore_barrier()

      @pl.when(c == 0)
      def _():
          run_pass(g0_hbm, 0)
          run_pass(g1_hbm, 1)

      @pl.when(c == 1)
      def _():
          run_pass(g2_hbm, 0)
          run_pass(g3_hbm, 1)


  return _gat_agg


def kernel(x, edges, lin1_w, lin1_b, lin2_w, lin2_b, lin3_w, lin3_b,
           bn1_g, bn1_b, bn2_g, bn2_b, bn3_g, bn3_b, bn4_g, bn4_b,
           bn5_g, bn5_b,
           gat1_w, gat1_asrc, gat1_adst, gat1_bias,
           gat2_w, gat2_asrc, gat2_adst, gat2_bias):
    del gat1_bias, gat2_bias
    loop = jnp.arange(N, dtype=jnp.int32)
    pad = jnp.arange(E_PAD - E_TOT, dtype=jnp.int32) % N
    src = jnp.concatenate([edges[0], loop, pad])
    dst = jnp.concatenate([edges[1], loop, pad])

    y1, st1 = _lin_stats(x, lin1_w, lin1_b)
    y2, st2 = _bn_lin(y1, st1, bn1_g, bn1_b, lin2_w, lin2_b)
    y3, st3 = _bn_lin(y2, st2, bn2_g, bn2_b, lin3_w, lin3_b)
    g10, g11, g12, g13, s1, d1 = _gat_pre(y3, st3, bn3_g, bn3_b, gat1_w,
                                          gat1_asrc, gat1_adst)
    agg1, stp1 = _make_gat_agg()(src, dst, s1.reshape(N), d1.reshape(N),
                          g10, g11, g12, g13)
    stp1 = jnp.transpose(stp1, (3, 2, 0, 1, 4)).reshape(2, NS, D)
    g20, g21, g22, g23, s2, d2 = _agg_gat_pre(agg1, stp1, bn4_g, bn4_b,
                                              gat2_w, gat2_asrc, gat2_adst)
    agg2, stp2 = _make_gat_agg()(src, dst, s2.reshape(N), d2.reshape(N),
                          g20, g21, g22, g23)
    stp2 = jnp.transpose(stp2, (3, 2, 0, 1, 4)).reshape(2, NS, D)
    return _final(agg2, stp2, bn5_g, bn5_b)


# speedup vs baseline: 17.5286x; 1.0512x over previous
import functools

import jax
import jax.numpy as jnp
from jax import lax
from jax.experimental import pallas as pl
from jax.experimental.pallas import tpu as pltpu
from jax.experimental.pallas import tpu_sc as plsc

N = 10000
D_IN = 2613
D = 256
DH = 128
DQ = 64
E = 160000
E_TOT = E + N
NC, NS, L = 2, 16, 16
EPT = 10752
E_PAD = EPT * NS
CH = 128
NCH = EPT // CH
NV = EPT // L
NPAD = 10240
RPT = NPAD // NS
ROW_TILE = 1000
GRID = N // ROW_TILE


def _bn_elu(y, st, gamma, beta):
    m = st[0:1, :] / N
    v = st[1:2, :] / N - m * m
    inv = lax.rsqrt(v + 1e-5)
    h = (y - m) * inv * gamma[None, :] + beta[None, :]
    return jnp.where(h > 0, h, jnp.exp(jnp.minimum(h, 0.0)) - 1.0)


def _mm_stats_body(x_ref, w_ref, b_ref, y_ref, st_ref):
    i = pl.program_id(0)
    y = lax.dot_general(x_ref[...].astype(jnp.bfloat16),
                        w_ref[...].astype(jnp.bfloat16),
                        (((1,), (1,)), ((), ())),
                        preferred_element_type=jnp.float32)
    y = y + b_ref[...][None, :]
    y_ref[...] = y

    @pl.when(i == 0)
    def _():
        st_ref[...] = jnp.zeros_like(st_ref)

    st_ref[0:1, :] = st_ref[0:1, :] + jnp.sum(y, axis=0, keepdims=True)
    st_ref[1:2, :] = st_ref[1:2, :] + jnp.sum(y * y, axis=0, keepdims=True)


def _lin_stats(x, w, b):
    k = x.shape[1]
    return pl.pallas_call(
        _mm_stats_body,
        grid=(GRID,),
        in_specs=[pl.BlockSpec((ROW_TILE, k), lambda i: (i, 0)),
                  pl.BlockSpec((D, k), lambda i: (0, 0)),
                  pl.BlockSpec((D,), lambda i: (0,))],
        out_specs=[pl.BlockSpec((ROW_TILE, D), lambda i: (i, 0)),
                   pl.BlockSpec((2, D), lambda i: (0, 0))],
        out_shape=[jax.ShapeDtypeStruct((x.shape[0], D), jnp.float32),
                   jax.ShapeDtypeStruct((2, D), jnp.float32)],
    )(x, w, b)


def _bn_lin_body(y_ref, st_ref, gam_ref, bet_ref, w_ref, b_ref, o_ref, sto_ref):
    i = pl.program_id(0)
    h = _bn_elu(y_ref[...], st_ref[...], gam_ref[...], bet_ref[...])
    y2 = lax.dot_general(h, w_ref[...], (((1,), (1,)), ((), ())),
                         preferred_element_type=jnp.float32)
    y2 = y2 + b_ref[...][None, :]
    o_ref[...] = y2

    @pl.when(i == 0)
    def _():
        sto_ref[...] = jnp.zeros_like(sto_ref)

    sto_ref[0:1, :] = sto_ref[0:1, :] + jnp.sum(y2, axis=0, keepdims=True)
    sto_ref[1:2, :] = sto_ref[1:2, :] + jnp.sum(y2 * y2, axis=0, keepdims=True)


def _bn_lin(y, st, gam, bet, w, b):
    return pl.pallas_call(
        _bn_lin_body,
        grid=(GRID,),
        in_specs=[pl.BlockSpec((ROW_TILE, D), lambda i: (i, 0)),
                  pl.BlockSpec((2, D), lambda i: (0, 0)),
                  pl.BlockSpec((D,), lambda i: (0,)),
                  pl.BlockSpec((D,), lambda i: (0,)),
                  pl.BlockSpec((D, D), lambda i: (0, 0)),
                  pl.BlockSpec((D,), lambda i: (0,))],
        out_specs=[pl.BlockSpec((ROW_TILE, D), lambda i: (i, 0)),
                   pl.BlockSpec((2, D), lambda i: (0, 0))],
        out_shape=[jax.ShapeDtypeStruct((N, D), jnp.float32),
                   jax.ShapeDtypeStruct((2, D), jnp.float32)],
    )(y, st, gam, bet, w, b)


def _gat_pre_body(y_ref, st_ref, gam_ref, bet_ref, w_ref, as_ref, ad_ref,
                  g0_ref, g1_ref, g2_ref, g3_ref, s_ref, d_ref):
    h = _bn_elu(y_ref[...], st_ref[...], gam_ref[...], bet_ref[...])
    g = lax.dot_general(h, w_ref[...], (((1,), (1,)), ((), ())),
                        preferred_element_type=jnp.float32)
    g0_ref[...] = g[:, 0 * DQ:1 * DQ]
    g1_ref[...] = g[:, 1 * DQ:2 * DQ]
    g2_ref[...] = g[:, 2 * DQ:3 * DQ]
    g3_ref[...] = g[:, 3 * DQ:4 * DQ]
    s_ref[...] = jnp.sum(g * as_ref[...][None, :], axis=1)[:, None]
    d_ref[...] = jnp.sum(g * ad_ref[...][None, :], axis=1)[:, None]


def _gat_pre(y, st, gam, bet, w, asrc, adst):
    return pl.pallas_call(
        _gat_pre_body,
        grid=(GRID,),
        in_specs=[pl.BlockSpec((ROW_TILE, D), lambda i: (i, 0)),
                  pl.BlockSpec((2, D), lambda i: (0, 0)),
                  pl.BlockSpec((D,), lambda i: (0,)),
                  pl.BlockSpec((D,), lambda i: (0,)),
                  pl.BlockSpec((D, D), lambda i: (0, 0)),
                  pl.BlockSpec((D,), lambda i: (0,)),
                  pl.BlockSpec((D,), lambda i: (0,))],
        out_specs=[pl.BlockSpec((ROW_TILE, DQ), lambda i: (i, 0)),
                   pl.BlockSpec((ROW_TILE, DQ), lambda i: (i, 0)),
                   pl.BlockSpec((ROW_TILE, DQ), lambda i: (i, 0)),
                   pl.BlockSpec((ROW_TILE, DQ), lambda i: (i, 0)),
                   pl.BlockSpec((ROW_TILE, 1), lambda i: (i, 0)),
                   pl.BlockSpec((ROW_TILE, 1), lambda i: (i, 0))],
        out_shape=[jax.ShapeDtypeStruct((N, DQ), jnp.float32),
                   jax.ShapeDtypeStruct((N, DQ), jnp.float32),
                   jax.ShapeDtypeStruct((N, DQ), jnp.float32),
                   jax.ShapeDtypeStruct((N, DQ), jnp.float32),
                   jax.ShapeDtypeStruct((N, 1), jnp.float32),
                   jax.ShapeDtypeStruct((N, 1), jnp.float32)],
    )(y, st, gam, bet, w, asrc, adst)


def _agg_gat_pre_body(a0_ref, a1_ref, a2_ref, a3_ref, stp_ref, gam_ref,
                      bet_ref, w_ref, as_ref, ad_ref,
                      g0_ref, g1_ref, g2_ref, g3_ref, s_ref, d_ref):
    st = jnp.sum(stp_ref[...], axis=1)
    y = jnp.concatenate([a0_ref[0, 0], a1_ref[0, 0], a2_ref[0, 0],
                         a3_ref[0, 0]], axis=1)
    h = _bn_elu(y, st, gam_ref[...], bet_ref[...])
    g = lax.dot_general(h, w_ref[...], (((1,), (1,)), ((), ())),
                        preferred_element_type=jnp.float32)
    g0_ref[...] = g[:, 0 * DQ:1 * DQ]
    g1_ref[...] = g[:, 1 * DQ:2 * DQ]
    g2_ref[...] = g[:, 2 * DQ:3 * DQ]
    g3_ref[...] = g[:, 3 * DQ:4 * DQ]
    s_ref[...] = jnp.sum(g * as_ref[...][None, :], axis=1)[:, None]
    d_ref[...] = jnp.sum(g * ad_ref[...][None, :], axis=1)[:, None]


def _agg_gat_pre(agg, stp, gam, bet, w, asrc, adst):
    return pl.pallas_call(
        _agg_gat_pre_body,
        grid=(GRID,),
        in_specs=[pl.BlockSpec((1, 1, ROW_TILE, DQ), lambda i: (0, 0, i, 0)),
                  pl.BlockSpec((1, 1, ROW_TILE, DQ), lambda i: (0, 1, i, 0)),
                  pl.BlockSpec((1, 1, ROW_TILE, DQ), lambda i: (1, 0, i, 0)),
                  pl.BlockSpec((1, 1, ROW_TILE, DQ), lambda i: (1, 1, i, 0)),
                  pl.BlockSpec((2, NS, D), lambda i: (0, 0, 0)),
                  pl.BlockSpec((D,), lambda i: (0,)),
                  pl.BlockSpec((D,), lambda i: (0,)),
                  pl.BlockSpec((D, D), lambda i: (0, 0)),
                  pl.BlockSpec((D,), lambda i: (0,)),
                  pl.BlockSpec((D,), lambda i: (0,))],
        out_specs=[pl.BlockSpec((ROW_TILE, DQ), lambda i: (i, 0)),
                   pl.BlockSpec((ROW_TILE, DQ), lambda i: (i, 0)),
                   pl.BlockSpec((ROW_TILE, DQ), lambda i: (i, 0)),
                   pl.BlockSpec((ROW_TILE, DQ), lambda i: (i, 0)),
                   pl.BlockSpec((ROW_TILE, 1), lambda i: (i, 0)),
                   pl.BlockSpec((ROW_TILE, 1), lambda i: (i, 0))],
        out_shape=[jax.ShapeDtypeStruct((N, DQ), jnp.float32),
                   jax.ShapeDtypeStruct((N, DQ), jnp.float32),
                   jax.ShapeDtypeStruct((N, DQ), jnp.float32),
                   jax.ShapeDtypeStruct((N, DQ), jnp.float32),
                   jax.ShapeDtypeStruct((N, 1), jnp.float32),
                   jax.ShapeDtypeStruct((N, 1), jnp.float32)],
    )(agg, agg, agg, agg, stp, gam, bet, w, asrc, adst)


def _final_body(a0_ref, a1_ref, a2_ref, a3_ref, stp_ref, gam_ref, bet_ref,
                o_ref):
    st = jnp.sum(stp_ref[...], axis=1)
    y = jnp.concatenate([a0_ref[0, 0], a1_ref[0, 0], a2_ref[0, 0],
                         a3_ref[0, 0]], axis=1)
    o_ref[...] = _bn_elu(y, st, gam_ref[...], bet_ref[...])


def _final(agg, stp, gam, bet):
    return pl.pallas_call(
        _final_body,
        grid=(GRID,),
        in_specs=[pl.BlockSpec((1, 1, ROW_TILE, DQ), lambda i: (0, 0, i, 0)),
                  pl.BlockSpec((1, 1, ROW_TILE, DQ), lambda i: (0, 1, i, 0)),
                  pl.BlockSpec((1, 1, ROW_TILE, DQ), lambda i: (1, 0, i, 0)),
                  pl.BlockSpec((1, 1, ROW_TILE, DQ), lambda i: (1, 1, i, 0)),
                  pl.BlockSpec((2, NS, D), lambda i: (0, 0, 0)),
                  pl.BlockSpec((D,), lambda i: (0,)),
                  pl.BlockSpec((D,), lambda i: (0,))],
        out_specs=pl.BlockSpec((ROW_TILE, D), lambda i: (i, 0)),
        out_shape=jax.ShapeDtypeStruct((N, D), jnp.float32),
    )(agg, agg, agg, agg, stp, gam, bet)


@functools.lru_cache(maxsize=1)
def _make_gat_agg():
  @functools.partial(
    pl.kernel,
    out_type=[jax.ShapeDtypeStruct((NC, 2, NPAD, DQ), jnp.float32),
              jax.ShapeDtypeStruct((NC, 2, NS, 2, DQ), jnp.float32)],
    mesh=plsc.VectorSubcoreMesh(core_axis_name="c", subcore_axis_name="s",
                                num_cores=NC, num_subcores=NS),
    compiler_params=pltpu.CompilerParams(needs_layout_passes=False,
                                         use_tc_tiling_on_sc=False),
    scratch_types=[
        pltpu.VMEM((EPT,), jnp.int32),
        pltpu.VMEM((EPT,), jnp.int32),
        pltpu.VMEM((N,), jnp.float32),
        pltpu.VMEM((N,), jnp.float32),
        pltpu.VMEM((EPT,), jnp.float32),
        pltpu.VMEM((CH, DQ), jnp.float32),
        pltpu.VMEM((CH, DQ), jnp.float32),
        pltpu.VMEM((CH, DQ), jnp.float32),
        pltpu.VMEM((CH, DQ), jnp.float32),
        pltpu.VMEM((CH,), jnp.int32),
        pltpu.VMEM((CH,), jnp.int32),
        pltpu.VMEM((CH,), jnp.float32),
        pltpu.VMEM((CH,), jnp.float32),
        pltpu.SemaphoreType.DMA,
        pltpu.SemaphoreType.DMA,
        pltpu.SemaphoreType.DMA,
        pltpu.SemaphoreType.DMA,
        pltpu.VMEM((RPT,), jnp.float32),
        pltpu.VMEM((L,), jnp.float32),
        pltpu.VMEM((NS, L), jnp.float32),
        pltpu.VMEM((2, DQ), jnp.float32),
        pltpu.VMEM_SHARED((NS, L), jnp.float32),
        pltpu.VMEM_SHARED((NPAD,), jnp.float32),
        pltpu.VMEM_SHARED((NPAD, DQ), jnp.float32),
      ],
  )
  def _gat_agg(src_hbm, dst_hbm, s_hbm, d_hbm, g0_hbm, g1_hbm, g2_hbm, g3_hbm,
               out_hbm, st_hbm,
               src_v, dst_v, s_loc, d_loc, a_v, buf, buf2, sbuf, sbuf2,
               idxw, idxw2, ew, ew2, gsem0, gsem1, ssem0, ssem1, den_r,
               maxbuf, max_l, statbuf, max_sh, den_sh, acc_sh):
      c = lax.axis_index("c")
      sid = lax.axis_index("s")
      base = sid * EPT
      rb = sid * RPT
      zv = jnp.zeros((L,), jnp.float32)
      NQV = DQ // L

      pltpu.sync_copy(src_hbm.at[pl.ds(base, EPT)], src_v)
      pltpu.sync_copy(dst_hbm.at[pl.ds(base, EPT)], dst_v)
      pltpu.sync_copy(s_hbm, s_loc)
      pltpu.sync_copy(d_hbm, d_loc)

      def alpha_body(i, m):
          si = src_v[pl.ds(i * L, L)]
          di = dst_v[pl.ds(i * L, L)]
          a = plsc.load_gather(s_loc, [si]) + plsc.load_gather(d_loc, [di])
          a = jnp.where(a > 0, a, 0.2 * a)
          eid = base + i * L + lax.iota(jnp.int32, L)
          a = jnp.where(eid < E_TOT, a, -1e30)
          a_v[pl.ds(i * L, L)] = a
          return jnp.maximum(m, a)

      m = lax.fori_loop(0, NV, alpha_body, jnp.full((L,), -1e30, jnp.float32))
      maxbuf[...] = m
      pltpu.sync_copy(maxbuf, max_sh.at[sid])

      def zero_den(i, _):
          den_r[pl.ds(i * L, L)] = zv
          return 0

      lax.fori_loop(0, RPT // L, zero_den, 0)
      pltpu.sync_copy(den_r, den_sh.at[pl.ds(rb, RPT)])


      plsc.subcore_barrier()

      pltpu.sync_copy(max_sh, max_l)

      def max_red(i, mm):
          return jnp.maximum(mm, max_l[i])

      m = lax.fori_loop(0, NS, max_red, m)
      gmax = jnp.max(m)

      ews = (ew, ew2)
      idxws_ = (idxw, idxw2)
      ssems_ = (ssem0, ssem1)

      def den_body(j2, _):
          for p in range(2):
              j = j2 * 2 + p
              eb, ib, ss = ews[p], idxws_[p], ssems_[p]

              @pl.when(j2 > 0)
              def _():
                  pltpu.make_async_copy(eb, den_sh.at[ib], ss).wait()

              for v in range(CH // L):
                  o = j * CH + v * L
                  e = jnp.exp(a_v[pl.ds(o, L)] - gmax)
                  a_v[pl.ds(o, L)] = e
                  eb[pl.ds(v * L, L)] = e
                  ib[pl.ds(v * L, L)] = dst_v[pl.ds(o, L)]
              pltpu.async_copy(eb, den_sh.at[ib], ss, add=True)
          return 0

      lax.fori_loop(0, NCH // 2, den_body, 0)
      for p in range(2):
          pltpu.make_async_copy(ews[p], den_sh.at[idxws_[p]], ssems_[p]).wait()
      plsc.subcore_barrier()

      pltpu.sync_copy(den_sh.at[pl.ds(rb, RPT)], den_r)

      def inv_body(i, _):
          dv = den_r[pl.ds(i * L, L)]
          den_r[pl.ds(i * L, L)] = 1.0 / (dv + 1e-16)
          return 0

      lax.fori_loop(0, RPT // L, inv_body, 0)

      bufs = (buf, buf2)
      sbufs = (sbuf, sbuf2)
      idxws = (idxw, idxw2)
      gsems = (gsem0, gsem1)
      ssems = (ssem0, ssem1)

      def zero_sbuf(r, _):
          for v in range(NQV):
              sbuf[r, pl.ds(v * L, L)] = zv
          return 0

      def run_pass(g_ref, qi):
          lax.fori_loop(0, CH, zero_sbuf, 0)
          for k in range(RPT // CH):
              pltpu.sync_copy(sbuf, acc_sh.at[pl.ds(rb + k * CH, CH)])
          plsc.subcore_barrier()

          pltpu.async_copy(g_ref.at[src_v.at[pl.ds(0, CH)]], buf, gsem0)

          def ch2_body(j2, _):
              for p in range(2):
                  j = j2 * 2 + p
                  b, sb, ib = bufs[p], sbufs[p], idxws[p]
                  gs, ss = gsems[p], ssems[p]

                  @pl.when(j2 > 0)
                  def _():
                      pltpu.make_async_copy(sb, acc_sh.at[ib], ss).wait()

                  @pl.when(j + 1 < NCH)
                  def _():
                      pltpu.async_copy(
                          g_ref.at[src_v.at[pl.ds((j + 1) * CH, CH)]],
                          bufs[1 - p], gsems[1 - p])

                  pltpu.make_async_copy(
                      g_ref.at[src_v.at[pl.ds(j * CH, CH)]], b, gs).wait()
                  for v in range(CH // L):
                      ib[pl.ds(v * L, L)] = dst_v[pl.ds(j * CH + v * L, L)]

                  def grp_body(gi, _):
                      wv = a_v[pl.ds(j * CH + gi * L, L)]
                      for r16 in range(L):
                          row = gi * L + r16
                          w = wv[r16]
                          for v in range(NQV):
                              sl = pl.ds(v * L, L)
                              sb[row, sl] = b[row, sl] * w
                      return 0

                  lax.fori_loop(0, CH // L, grp_body, 0)
                  pltpu.async_copy(sb, acc_sh.at[ib], ss, add=True)
              return 0

          lax.fori_loop(0, NCH // 2, ch2_body, 0)
          for p in range(2):
              pltpu.make_async_copy(sbufs[p], acc_sh.at[idxws[p]],
                                    ssems[p]).wait()
          plsc.subcore_barrier()

          ssum = tuple(zv for _ in range(NQV))
          ssq = tuple(zv for _ in range(NQV))
          NKO = RPT // CH
          pltpu.async_copy(acc_sh.at[pl.ds(rb, CH)], buf, gsem0)
          for k in range(NKO):
              p = k % 2
              b, sb, gs, ss = bufs[p], sbufs[p], gsems[p], ssems[p]
              if k + 1 < NKO:
                  pltpu.async_copy(acc_sh.at[pl.ds(rb + (k + 1) * CH, CH)],
                                   bufs[1 - p], gsems[1 - p])
              pltpu.make_async_copy(acc_sh.at[pl.ds(rb + k * CH, CH)],
                                    b, gs).wait()
              if k >= 2:
                  pltpu.make_async_copy(
                      sb, out_hbm.at[c, qi, pl.ds(rb + (k - 2) * CH, CH)],
                      ss).wait()

              def grp_out(gi, carry):
                  su, sq = list(carry[0]), list(carry[1])
                  iv = den_r[pl.ds(k * CH + gi * L, L)]
                  for r16 in range(L):
                      row = gi * L + r16
                      inv = iv[r16]
                      for v in range(NQV):
                          sl = pl.ds(v * L, L)
                          x = b[row, sl] * inv
                          sb[row, sl] = x
                          su[v] = su[v] + x
                          sq[v] = sq[v] + x * x
                  return tuple(su), tuple(sq)

              ssum, ssq = lax.fori_loop(0, CH // L, grp_out, (ssum, ssq))
              pltpu.async_copy(sb, out_hbm.at[c, qi, pl.ds(rb + k * CH, CH)],
                               ss)
          for k in range(max(0, NKO - 2), NKO):
              p = k % 2
              pltpu.make_async_copy(
                  sbufs[p], out_hbm.at[c, qi, pl.ds(rb + k * CH, CH)],
                  ssems[p]).wait()
          for v in range(NQV):
              statbuf[0, pl.ds(v * L, L)] = ssum[v]
              statbuf[1, pl.ds(v * L, L)] = ssq[v]
          pltpu.sync_copy(statbuf, st_hbm.at[c, qi, sid])
          plsc.subcore_barrier()

      @pl.when(c == 0)
      def _():
          run_pass(g0_hbm, 0)
          run_pass(g1_hbm, 1)

      @pl.when(c == 1)
      def _():
          run_pass(g2_hbm, 0)
          run_pass(g3_hbm, 1)


  return _gat_agg


def kernel(x, edges, lin1_w, lin1_b, lin2_w, lin2_b, lin3_w, lin3_b,
           bn1_g, bn1_b, bn2_g, bn2_b, bn3_g, bn3_b, bn4_g, bn4_b,
           bn5_g, bn5_b,
           gat1_w, gat1_asrc, gat1_adst, gat1_bias,
           gat2_w, gat2_asrc, gat2_adst, gat2_bias):
    del gat1_bias, gat2_bias
    loop = jnp.arange(N, dtype=jnp.int32)
    pad = jnp.arange(E_PAD - E_TOT, dtype=jnp.int32) % N
    src = jnp.concatenate([edges[0], loop, pad])
    dst = jnp.concatenate([edges[1], loop, pad])

    y1, st1 = _lin_stats(x, lin1_w, lin1_b)
    y2, st2 = _bn_lin(y1, st1, bn1_g, bn1_b, lin2_w, lin2_b)
    y3, st3 = _bn_lin(y2, st2, bn2_g, bn2_b, lin3_w, lin3_b)
    g10, g11, g12, g13, s1, d1 = _gat_pre(y3, st3, bn3_g, bn3_b, gat1_w,
                                          gat1_asrc, gat1_adst)
    agg1, stp1 = _make_gat_agg()(src, dst, s1.reshape(N), d1.reshape(N),
                          g10, g11, g12, g13)
    stp1 = jnp.transpose(stp1, (3, 2, 0, 1, 4)).reshape(2, NS, D)
    g20, g21, g22, g23, s2, d2 = _agg_gat_pre(agg1, stp1, bn4_g, bn4_b,
                                              gat2_w, gat2_asrc, gat2_adst)
    agg2, stp2 = _make_gat_agg()(src, dst, s2.reshape(N), d2.reshape(N),
                          g20, g21, g22, g23)
    stp2 = jnp.transpose(stp2, (3, 2, 0, 1, 4)).reshape(2, NS, D)
    return _final(agg2, stp2, bn5_g, bn5_b)
